# baseline (speedup 1.0000x reference)
import jax
import jax.numpy as jnp
from jax.experimental import pallas as pl
from jax.experimental.pallas import tpu as pltpu

D_MODEL_ = 1024
NUM_EXPERTS_ = 8
D_FF_ = 2048


def _moe_kernel(x_ref, rw_ref, w1_hbm, w2_hbm, out_ref,
                w1_buf, w2_buf, s1, s2):
    e = pl.program_id(0)
    x = x_ref[...]

    hf = D_FF_ // 2
    hd = D_MODEL_ // 2

    def start_copies(src_e, slot):
        pltpu.make_async_copy(w1_hbm.at[src_e, pl.ds(0, hf)],
                              w1_buf.at[slot, pl.ds(0, hf)],
                              s1.at[slot, 0]).start()
        pltpu.make_async_copy(w1_hbm.at[src_e, pl.ds(hf, hf)],
                              w1_buf.at[slot, pl.ds(hf, hf)],
                              s1.at[slot, 1]).start()
        pltpu.make_async_copy(w2_hbm.at[src_e, pl.ds(0, hd)],
                              w2_buf.at[slot, pl.ds(0, hd)],
                              s2.at[slot, 0]).start()
        pltpu.make_async_copy(w2_hbm.at[src_e, pl.ds(hd, hd)],
                              w2_buf.at[slot, pl.ds(hd, hd)],
                              s2.at[slot, 1]).start()

    @pl.when(e == 0)
    def _():
        start_copies(0, 0)

    @pl.when(e < NUM_EXPERTS_ - 1)
    def _():
        nxt = (e + 1) % 2

        @pl.when(nxt == 1)
        def _():
            start_copies(e + 1, 1)

        @pl.when(nxt == 0)
        def _():
            start_copies(e + 1, 0)

    logits = jax.lax.dot_general(
        x, rw_ref[...], (((1,), (1,)), ((), ())),
        preferred_element_type=jnp.float32)
    col = jax.lax.broadcasted_iota(jnp.int32, logits.shape, 1)
    m1 = jnp.max(logits, axis=1, keepdims=True)
    i1 = jnp.min(jnp.where(logits == m1, col, NUM_EXPERTS_), axis=1,
                 keepdims=True)
    masked = jnp.where(col == i1, -jnp.inf, logits)
    m2 = jnp.max(masked, axis=1, keepdims=True)
    i2 = jnp.min(jnp.where(masked == m2, col, NUM_EXPERTS_), axis=1,
                 keepdims=True)
    p1 = 1.0 / (1.0 + jnp.exp(m2 - m1))
    p2 = 1.0 - p1
    c_e = jnp.sum(jnp.where(col == i1, p1, 0.0) * (col == e)
                  + jnp.where(col == i2, p2, 0.0) * (col == e), axis=1)

    def gelu(t):
        return 0.5 * t * (1.0 + jax.lax.erf(t * (2.0 ** -0.5)))

    def ffn(slot):
        pltpu.make_async_copy(w1_hbm.at[e, pl.ds(0, hf)],
                              w1_buf.at[slot, pl.ds(0, hf)],
                              s1.at[slot, 0]).wait()
        h_a = gelu(jax.lax.dot_general(
            x, w1_buf[slot, :hf, :], (((1,), (1,)), ((), ())),
            preferred_element_type=jnp.float32))
        pltpu.make_async_copy(w1_hbm.at[e, pl.ds(hf, hf)],
                              w1_buf.at[slot, pl.ds(hf, hf)],
                              s1.at[slot, 1]).wait()
        h_b = gelu(jax.lax.dot_general(
            x, w1_buf[slot, hf:, :], (((1,), (1,)), ((), ())),
            preferred_element_type=jnp.float32))
        h = jnp.concatenate([h_a, h_b], axis=1)

        pltpu.make_async_copy(w2_hbm.at[e, pl.ds(0, hd)],
                              w2_buf.at[slot, pl.ds(0, hd)],
                              s2.at[slot, 0]).wait()
        y_a = jax.lax.dot_general(h, w2_buf[slot, :hd, :],
                                  (((1,), (1,)), ((), ())),
                                  preferred_element_type=jnp.float32)
        ca = c_e[:, None] * y_a

        @pl.when(e == 0)
        def _():
            out_ref[:, :hd] = ca

        @pl.when(e > 0)
        def _():
            out_ref[:, :hd] += ca

        pltpu.make_async_copy(w2_hbm.at[e, pl.ds(hd, hd)],
                              w2_buf.at[slot, pl.ds(hd, hd)],
                              s2.at[slot, 1]).wait()
        y_b = jax.lax.dot_general(h, w2_buf[slot, hd:, :],
                                  (((1,), (1,)), ((), ())),
                                  preferred_element_type=jnp.float32)
        cb = c_e[:, None] * y_b

        @pl.when(e == 0)
        def _():
            out_ref[:, hd:] = cb

        @pl.when(e > 0)
        def _():
            out_ref[:, hd:] += cb

    @pl.when(e % 2 == 0)
    def _():
        ffn(0)

    @pl.when(e % 2 == 1)
    def _():
        ffn(1)


@jax.jit
def _moe(x_flat, w1, w2, router_w):
    n = x_flat.shape[0]
    return pl.pallas_call(
        _moe_kernel,
        grid=(NUM_EXPERTS_,),
        in_specs=[
            pl.BlockSpec((n, D_MODEL_), lambda e: (0, 0)),
            pl.BlockSpec((NUM_EXPERTS_, D_MODEL_), lambda e: (0, 0)),
            pl.BlockSpec(memory_space=pl.ANY),
            pl.BlockSpec(memory_space=pl.ANY),
        ],
        out_specs=pl.BlockSpec((n, D_MODEL_), lambda e: (0, 0)),
        out_shape=jax.ShapeDtypeStruct((n, D_MODEL_), jnp.float32),
        scratch_shapes=[
            pltpu.VMEM((2, D_FF_, D_MODEL_), jnp.float32),
            pltpu.VMEM((2, D_MODEL_, D_FF_), jnp.float32),
            pltpu.SemaphoreType.DMA((2, 2)),
            pltpu.SemaphoreType.DMA((2, 2)),
        ],
    )(x_flat, router_w, w1, w2)


def kernel(x, w1, w2, router_w):
    B, T, D = x.shape
    out = _moe(x.reshape(B * T, D), w1, w2, router_w)
    return out.reshape(B, T, D)
